# baseline (speedup 1.0000x reference)
import jax
import jax.numpy as jnp
from jax import lax
from jax.experimental import pallas as pl
from jax.experimental.pallas import tpu as pltpu
from jax.experimental.pallas import tpu_sc as plsc

_EMB = 32
_LANES = 16
_NUM_WORKERS = 32


def _neg_norm(acc):
    x = jnp.maximum(acc, jnp.float32(1e-30))
    xi = lax.bitcast_convert_type(x, jnp.int32)
    yi = jnp.int32(0x5F3759DF) - lax.shift_right_logical(xi, 1)
    y = lax.bitcast_convert_type(yi, jnp.float32)
    half = x * jnp.float32(0.5)
    for _ in range(3):
        y = y * (jnp.float32(1.5) - half * y * y)
    return -(x * y)


def _make_body(b_per_w):
    n_groups = b_per_w // _LANES

    def body(heads_hbm, rels_hbm, tails_hbm, ent_hbm, rel_hbm, out_hbm,
             hidx, ridx, tidx, hrows, rrows, trows, outv, sem):
        wid = lax.axis_index("s") * 2 + lax.axis_index("c")
        base = wid * b_per_w

        pltpu.sync_copy(heads_hbm.at[pl.ds(base, b_per_w)], hidx)
        pltpu.sync_copy(rels_hbm.at[pl.ds(base, b_per_w)], ridx)
        pltpu.sync_copy(tails_hbm.at[pl.ds(base, b_per_w)], tidx)

        ch = pltpu.async_copy(ent_hbm.at[hidx], hrows, sem)
        cr = pltpu.async_copy(rel_hbm.at[ridx], rrows, sem)
        ct = pltpu.async_copy(ent_hbm.at[tidx], trows, sem)
        ch.wait()
        cr.wait()
        ct.wait()

        lane = lax.iota(jnp.int32, _LANES)

        def group(g, carry):
            row = g * _LANES + lane
            acc = jnp.zeros((_LANES,), jnp.float32)
            for j in range(_EMB):
                col = jnp.full((_LANES,), j, jnp.int32)
                hv = plsc.load_gather(hrows, [row, col])
                rv = plsc.load_gather(rrows, [row, col])
                tv = plsc.load_gather(trows, [row, col])
                d = (hv + rv) - tv
                acc = acc + d * d
            outv[pl.ds(g * _LANES, _LANES)] = _neg_norm(acc)
            return carry

        lax.fori_loop(0, n_groups, group, 0)
        pltpu.sync_copy(outv, out_hbm.at[pl.ds(base, b_per_w)])

    return body


def kernel(heads, rels, tails, ent_embs, rel_embs):
    batch = heads.shape[0]
    b_per_w = batch // _NUM_WORKERS

    sc_kernel = pl.kernel(
        _make_body(b_per_w),
        out_type=jax.ShapeDtypeStruct((batch,), jnp.float32),
        mesh=plsc.VectorSubcoreMesh(core_axis_name="c", subcore_axis_name="s"),
        scratch_types=[
            pltpu.VMEM((b_per_w,), jnp.int32),
            pltpu.VMEM((b_per_w,), jnp.int32),
            pltpu.VMEM((b_per_w,), jnp.int32),
            pltpu.VMEM((b_per_w, _EMB), jnp.float32),
            pltpu.VMEM((b_per_w, _EMB), jnp.float32),
            pltpu.VMEM((b_per_w, _EMB), jnp.float32),
            pltpu.VMEM((b_per_w,), jnp.float32),
            pltpu.SemaphoreType.DMA,
        ],
        compiler_params=pltpu.CompilerParams(
            needs_layout_passes=False, use_tc_tiling_on_sc=False
        ),
    )
    return sc_kernel(heads, rels, tails, ent_embs, rel_embs)
